# baseline (speedup 1.0000x reference)
import jax
import jax.numpy as jnp
from jax.experimental import pallas as pl
from jax.experimental.pallas import tpu as pltpu

THRESH = 0.5
ALPHA = 0.1

_ROWS_PER_BLK = 10816


def _body(x_ref, y_ref, o_ref):
    @pl.when(pl.program_id(0) == 0)
    def _():
        o_ref[0, 0] = jnp.float32(0.0)

    xv = x_ref[...]
    yv = y_ref[...]
    m = (yv[:, 0:1] > THRESH).astype(jnp.float32)
    d = (yv - xv) ** 2 - ALPHA * xv * xv
    s = jnp.sum(d * m) + ALPHA * jnp.sum(xv[:, 0] ** 2)
    o_ref[0, 0] += s


def kernel(x, y):
    B, N, C = x.shape
    R = B * N
    x2 = x.reshape(R, C)
    y2 = y.reshape(R, C)
    nblk = _ROWS_PER_BLK
    grid = (R // nblk,)
    out = pl.pallas_call(
        _body,
        grid=grid,
        in_specs=[
            pl.BlockSpec((nblk, C), lambda i: (i, 0)),
            pl.BlockSpec((nblk, C), lambda i: (i, 0)),
        ],
        out_specs=pl.BlockSpec((1, 1), lambda i: (0, 0),
                               memory_space=pltpu.SMEM),
        out_shape=jax.ShapeDtypeStruct((1, 1), jnp.float32),
    )(x2, y2)
    return out[0, 0]


# speedup vs baseline: 1.9485x; 1.9485x over previous
import jax
import jax.numpy as jnp
from jax.experimental import pallas as pl
from jax.experimental.pallas import tpu as pltpu

THRESH = 0.5
ALPHA = 0.1

def _body(x_ref, y_ref, o_ref):
    @pl.when(pl.program_id(0) == 0)
    def _():
        o_ref[0, 0] = jnp.float32(0.0)

    xv = x_ref[0]
    yv = y_ref[0]
    m = (yv[:, 0:1] > THRESH).astype(jnp.float32)
    d = (yv - xv) ** 2 - ALPHA * xv * xv
    s = jnp.sum(d * m) + ALPHA * jnp.sum(xv[:, 0] ** 2)
    o_ref[0, 0] += s


def kernel(x, y):
    B, N, C = x.shape
    out = pl.pallas_call(
        _body,
        grid=(B,),
        in_specs=[
            pl.BlockSpec((1, N, C), lambda i: (i, 0, 0)),
            pl.BlockSpec((1, N, C), lambda i: (i, 0, 0)),
        ],
        out_specs=pl.BlockSpec((1, 1), lambda i: (0, 0),
                               memory_space=pltpu.SMEM),
        out_shape=jax.ShapeDtypeStruct((1, 1), jnp.float32),
    )(x, y)
    return out[0, 0]
